# baseline (speedup 1.0000x reference)
import jax
import jax.numpy as jnp
from jax import lax
from jax.experimental import pallas as pl
from jax.experimental.pallas import tpu as pltpu
import jax.experimental.pallas.tpu_sc as plsc

B = 16
SEQ = 77
N_CTX = 16
N_VERB = 64
CTX_DIM = 512

PAD_SEQ = 96
V_PER_W = N_VERB // 2
LANES = 16
G_ROWS = 80


def _sc_body(tok_hbm, nn_hbm, pre_hbm, table_hbm, ctx_hbm,
             out_hbm, ct_hbm,
             tok_v, nn_v, pre_v, idx_v, ct_v, slab_a, slab_b, cbuf_a, cbuf_b,
             sem_a, sem_b, gsem):
    c = lax.axis_index("c")
    s = lax.axis_index("s")
    wid = s * 2 + c
    b = wid // 2
    h = wid % 2
    v0 = h * V_PER_W

    pltpu.sync_copy(tok_hbm.at[pl.ds(b * PAD_SEQ, PAD_SEQ)], tok_v)
    pltpu.sync_copy(nn_hbm, nn_v)
    pltpu.sync_copy(pre_hbm, pre_v)

    bvec = jnp.full((LANES,), b, jnp.int32)
    n1 = plsc.load_gather(nn_v, [bvec])
    n = jnp.max(n1)
    iota = lax.iota(jnp.int32, LANES)

    for ci in range(G_ROWS // LANES):
        j = LANES * ci + iota
        in_ctx = (j > n1) & (j <= n1 + N_CTX)
        tidx = jnp.where(j <= n1, j, j - N_CTX)
        tidx = jnp.clip(tidx, 0, SEQ - 1)
        tok = plsc.load_gather(tok_v, [tidx])
        cidx = jnp.clip(j - 1 - n1, 0, N_CTX - 1)
        pre = plsc.load_gather(pre_v, [cidx])
        ct_v[pl.ds(LANES * ci, LANES)] = jnp.where(in_ctx, pre, tok)
        idx_v[pl.ds(LANES * ci, LANES)] = tok

    @pl.when(h == 0)
    def _():
        pltpu.sync_copy(ct_v, ct_hbm.at[pl.ds(b * PAD_SEQ, G_ROWS)])

    pltpu.async_copy(table_hbm.at[idx_v], slab_a, gsem)
    pltpu.async_copy(table_hbm.at[idx_v], slab_b, gsem)
    pltpu.make_async_copy(table_hbm.at[idx_v], slab_a, gsem).wait()
    pltpu.make_async_copy(table_hbm.at[idx_v], slab_b, gsem).wait()

    def _splice_ctx(slab, cbuf):
        def _row(w, carry):
            row = n + 1 + w
            for cc in range(CTX_DIM // LANES):
                slab[row, pl.ds(cc * LANES, LANES)] = (
                    cbuf[w, pl.ds(cc * LANES, LANES)])
            return carry
        lax.fori_loop(0, N_CTX, _row, 0)

    pltpu.sync_copy(ctx_hbm.at[v0], cbuf_a)
    _splice_ctx(slab_a, cbuf_a)
    pltpu.async_copy(slab_a, out_hbm.at[b, v0], sem_a)
    pltpu.sync_copy(ctx_hbm.at[v0 + 1], cbuf_b)
    _splice_ctx(slab_b, cbuf_b)
    pltpu.async_copy(slab_b, out_hbm.at[b, v0 + 1], sem_b)

    def _step(g, carry):
        v = v0 + 2 * g
        pltpu.sync_copy(ctx_hbm.at[v], cbuf_a)
        pltpu.make_async_copy(slab_a, out_hbm.at[b, v], sem_a).wait()
        _splice_ctx(slab_a, cbuf_a)
        pltpu.async_copy(slab_a, out_hbm.at[b, v], sem_a)
        pltpu.sync_copy(ctx_hbm.at[v + 1], cbuf_b)
        pltpu.make_async_copy(slab_b, out_hbm.at[b, v + 1], sem_b).wait()
        _splice_ctx(slab_b, cbuf_b)
        pltpu.async_copy(slab_b, out_hbm.at[b, v + 1], sem_b)
        return carry

    lax.fori_loop(1, V_PER_W // 2, _step, 0)

    v_last = v0 + V_PER_W - 2
    pltpu.make_async_copy(slab_a, out_hbm.at[b, v_last], sem_a).wait()
    pltpu.make_async_copy(slab_b, out_hbm.at[b, v_last + 1], sem_b).wait()


def _sc_stage(tok_pad, nn, prefix, table, ctx):
    mesh = plsc.VectorSubcoreMesh(core_axis_name="c", subcore_axis_name="s",
                                  num_cores=2, num_subcores=16)
    sc_fn = pl.kernel(
        _sc_body,
        out_type=(
            jax.ShapeDtypeStruct((B, N_VERB, G_ROWS, CTX_DIM), jnp.float32),
            jax.ShapeDtypeStruct((B * PAD_SEQ,), jnp.int32),
        ),
        mesh=mesh,
        compiler_params=pltpu.CompilerParams(needs_layout_passes=False),
        scratch_types=[
            pltpu.VMEM((PAD_SEQ,), jnp.int32),
            pltpu.VMEM((16,), jnp.int32),
            pltpu.VMEM((N_CTX,), jnp.int32),
            pltpu.VMEM((G_ROWS,), jnp.int32),
            pltpu.VMEM((G_ROWS,), jnp.int32),
            pltpu.VMEM((G_ROWS, CTX_DIM), jnp.float32),
            pltpu.VMEM((G_ROWS, CTX_DIM), jnp.float32),
            pltpu.VMEM((N_CTX, CTX_DIM), jnp.float32),
            pltpu.VMEM((N_CTX, CTX_DIM), jnp.float32),
            pltpu.SemaphoreType.DMA,
            pltpu.SemaphoreType.DMA,
            pltpu.SemaphoreType.DMA,
        ],
    )
    return sc_fn(tok_pad, nn, prefix, table, ctx)


@jax.jit
def kernel(nouns_token, nouns_numbers, ctx, token_embedding_weight,
           prompt_prefix_token):
    tok_pad = jnp.zeros((B, PAD_SEQ), jnp.int32).at[:, :SEQ].set(nouns_token)
    prefix = prompt_prefix_token.reshape(N_CTX).astype(jnp.int32)
    nn = nouns_numbers.astype(jnp.int32)

    out80, ct_flat = _sc_stage(tok_pad.reshape(B * PAD_SEQ), nn, prefix,
                               token_embedding_weight, ctx)
    return out80[:, :, :SEQ, :], ct_flat.reshape(B, PAD_SEQ)[:, :SEQ]
